# baseline (speedup 1.0000x reference)
import functools

import jax
import jax.numpy as jnp
from jax import lax
from jax.experimental import pallas as pl
from jax.experimental.pallas import tpu as pltpu
from jax.experimental.pallas import tpu_sc as plsc

N_NODES = 10000
N_EDGES = 160000
D_IN = 767
D_H = 16
D_OUT = 10

NODES_P = 10240
E_TOT = N_EDGES + N_NODES
NW = 32
CH = 128
EPW = 5376
E_PAD = EPW * NW
NCH = EPW // CH
RPS = NODES_P // 16

ROW_BLK = 2048
N_BLKS = NODES_P // ROW_BLK

_sc_mesh = functools.partial(
    plsc.VectorSubcoreMesh, core_axis_name="c", subcore_axis_name="s")
_sc_params = pltpu.CompilerParams(use_tc_tiling_on_sc=False)


def _deg_body(dst_hbm, out_hbm, acc, zbuf, ones, didx, isem, ssem, csem):
  cid = lax.axis_index("c")
  sid = lax.axis_index("s")
  wid = sid * 2 + cid
  pltpu.async_copy(dst_hbm.at[wid], didx, isem)
  zero16 = jnp.zeros((16,), jnp.float32)
  one16 = jnp.ones((16,), jnp.float32)
  for i in range(RPS // 16):
    zbuf[pl.ds(i * 16, 16)] = zero16
  for i in range(CH // 16):
    ones[pl.ds(i * 16, 16)] = one16
  pltpu.sync_copy(zbuf, acc.at[pl.ds(sid * RPS, RPS)])
  pltpu.make_async_copy(dst_hbm.at[wid], didx, isem).wait()
  plsc.subcore_barrier()

  def fire(k, carry):
    pltpu.async_copy(ones, acc.at[didx.at[k]], ssem, add=True)
    return carry

  lax.fori_loop(0, NCH, fire, 0)

  def drain(k, carry):
    pltpu.make_async_copy(ones, acc.at[didx.at[k]], ssem).wait()
    return carry

  lax.fori_loop(0, NCH, drain, 0)
  plsc.subcore_barrier()
  pltpu.async_copy(acc.at[pl.ds(sid * RPS, RPS)],
                   out_hbm.at[cid, pl.ds(sid * RPS, RPS)], csem).wait()


def _deg_call(dst3):
  return pl.kernel(
      _deg_body,
      out_type=jax.ShapeDtypeStruct((2, NODES_P), jnp.float32),
      mesh=_sc_mesh(),
      compiler_params=_sc_params,
      scratch_types=[
          pltpu.VMEM_SHARED((NODES_P,), jnp.float32),
          pltpu.VMEM((RPS,), jnp.float32),
          pltpu.VMEM((CH,), jnp.float32),
          pltpu.VMEM((NCH, CH), jnp.int32),
          pltpu.SemaphoreType.DMA,
          pltpu.SemaphoreType.DMA,
          pltpu.SemaphoreType.DMA,
      ],
  )(dst3)


_NB = 7
_LAG = 3


def _edge_body(y_hbm, src_hbm, dst_hbm, out_hbm, acc, zbuf, sidx, didx,
               *rest):
  rows = list(rest[:_NB])
  gsem = list(rest[_NB:2 * _NB])
  ssem = list(rest[2 * _NB:3 * _NB])
  isem, zsem, csem = rest[3 * _NB:]
  cid = lax.axis_index("c")
  sid = lax.axis_index("s")
  wid = sid * 2 + cid
  pltpu.async_copy(src_hbm.at[wid], sidx, isem)
  pltpu.async_copy(dst_hbm.at[wid], didx, isem)
  zero16 = jnp.zeros((16,), jnp.float32)
  for i in range(64):
    zbuf[i, :] = zero16
  for k in range(RPS // 64):
    pltpu.async_copy(zbuf, acc.at[pl.ds(sid * RPS + k * 64, 64)], zsem)
  for k in range(RPS // 64):
    pltpu.make_async_copy(zbuf, acc.at[pl.ds(sid * RPS + k * 64, 64)],
                          zsem).wait()
  pltpu.make_async_copy(src_hbm.at[wid], sidx, isem).wait()
  pltpu.make_async_copy(dst_hbm.at[wid], didx, isem).wait()
  plsc.subcore_barrier()

  def gather(kk, b):
    pltpu.async_copy(y_hbm.at[sidx.at[kk]], rows[b], gsem[b])

  def wait_gather(kk, b):
    pltpu.make_async_copy(y_hbm.at[sidx.at[kk]], rows[b], gsem[b]).wait()

  def scatter(kk, b):
    pltpu.async_copy(rows[b], acc.at[didx.at[kk]], ssem[b], add=True)

  def wait_scatter(kk, b):
    pltpu.make_async_copy(rows[b], acc.at[didx.at[kk]], ssem[b]).wait()

  for kk in range(_NB):
    gather(kk, kk)
    if kk >= _LAG:
      wait_gather(kk - _LAG, kk - _LAG)
      scatter(kk - _LAG, kk - _LAG)

  def steady(i, carry):
    kbase = i * _NB
    for b in range(_NB):
      kk = kbase + b
      wait_scatter(kk - _NB, b)
      gather(kk, b)
      bj = (b - _LAG) % _NB
      wait_gather(kk - _LAG, bj)
      scatter(kk - _LAG, bj)
    return carry

  lax.fori_loop(1, NCH // _NB, steady, 0)

  for j in range(NCH - _LAG, NCH):
    wait_gather(j, j % _NB)
    scatter(j, j % _NB)
  for kk in range(NCH - _NB, NCH):
    wait_scatter(kk, kk % _NB)

  plsc.subcore_barrier()
  pltpu.async_copy(acc.at[pl.ds(sid * RPS, RPS)],
                   out_hbm.at[cid, pl.ds(sid * RPS, RPS)], csem).wait()


def _edge_call(y_pad, src3, dst3):
  return pl.kernel(
      _edge_body,
      out_type=jax.ShapeDtypeStruct((2, NODES_P, D_H), jnp.float32),
      mesh=_sc_mesh(),
      compiler_params=_sc_params,
      scratch_types=(
          [pltpu.VMEM_SHARED((NODES_P, D_H), jnp.float32),
           pltpu.VMEM((64, D_H), jnp.float32),
           pltpu.VMEM((NCH, CH), jnp.int32),
           pltpu.VMEM((NCH, CH), jnp.int32)]
          + [pltpu.VMEM((CH, D_H), jnp.float32)] * _NB
          + [pltpu.SemaphoreType.DMA] * (2 * _NB + 3)
      ),
  )(y_pad, src3, dst3)


def _dinv_whole(degp_ref, lo, size):
  deg = (degp_ref[0, pl.ds(lo, size)] + degp_ref[1, pl.ds(lo, size)])
  return jnp.where(deg > 0, lax.rsqrt(deg), 0.0)


def _mm1_body(x_ref, w_ref, degp_ref, y_ref):
  i = pl.program_id(0)
  xw = jnp.dot(x_ref[...], w_ref[...], preferred_element_type=jnp.float32)
  dinv = _dinv_whole(degp_ref, i * ROW_BLK, ROW_BLK)
  y_ref[...] = xw * dinv[:, None]


def _mm1_call(x, w1, degp):
  return pl.pallas_call(
      _mm1_body,
      grid=(N_BLKS,),
      in_specs=[
          pl.BlockSpec((ROW_BLK, D_IN), lambda i: (i, 0)),
          pl.BlockSpec((D_IN, D_H), lambda i: (0, 0)),
          pl.BlockSpec((2, NODES_P), lambda i: (0, 0)),
      ],
      out_specs=pl.BlockSpec((ROW_BLK, D_H), lambda i: (i, 0)),
      out_shape=jax.ShapeDtypeStruct((NODES_P, D_H), jnp.float32),
  )(x, w1, degp)


def _mm2_body(p_ref, degp_ref, w2_ref, b1_ref, y_ref):
  dinv = _dinv_whole(degp_ref, 0, NODES_P)
  acc = p_ref[0] + p_ref[1]
  h = jax.nn.relu(acc * dinv[:, None] + b1_ref[0, :])
  y_ref[...] = jnp.dot(h, w2_ref[...],
                       preferred_element_type=jnp.float32) * dinv[:, None]


def _mm2_call(p, degp, w2p, b1):
  return pl.pallas_call(
      _mm2_body,
      in_specs=[
          pl.BlockSpec((2, NODES_P, D_H), lambda: (0, 0, 0)),
          pl.BlockSpec((2, NODES_P), lambda: (0, 0)),
          pl.BlockSpec((D_H, D_H), lambda: (0, 0)),
          pl.BlockSpec((1, D_H), lambda: (0, 0)),
      ],
      out_specs=pl.BlockSpec((NODES_P, D_H), lambda: (0, 0)),
      out_shape=jax.ShapeDtypeStruct((NODES_P, D_H), jnp.float32),
  )(p, degp, w2p, b1)


def _post_body(q_ref, degp_ref, b2_ref, out_ref):
  dinv = _dinv_whole(degp_ref, 0, NODES_P)
  res = (q_ref[0] + q_ref[1]) * dinv[:, None] + b2_ref[0, :]
  out_ref[...] = res[:N_NODES, :D_OUT]


def _post_call(q, degp, b2p):
  return pl.pallas_call(
      _post_body,
      in_specs=[
          pl.BlockSpec((2, NODES_P, D_H), lambda: (0, 0, 0)),
          pl.BlockSpec((2, NODES_P), lambda: (0, 0)),
          pl.BlockSpec((1, D_H), lambda: (0, 0)),
      ],
      out_specs=pl.BlockSpec((N_NODES, D_OUT), lambda: (0, 0)),
      out_shape=jax.ShapeDtypeStruct((N_NODES, D_OUT), jnp.float32),
  )(q, degp, b2p)


def kernel(x, edge_index, W1, b1, W2, b2):
  n = x.shape[0]
  loop = jnp.arange(n, dtype=jnp.int32)
  ed = jnp.concatenate(
      [edge_index,
       jnp.tile(loop[None], (2, 1)),
       jnp.full((2, E_PAD - E_TOT), n, dtype=jnp.int32)],
      axis=1).reshape(2, NW, NCH, CH)
  src3, dst3 = ed[0], ed[1]

  w2p = jnp.zeros((D_H, D_H), jnp.float32).at[:, :D_OUT].set(W2)
  b1r = b1.reshape(1, D_H)
  b2p = jnp.zeros((1, D_H), jnp.float32).at[0, :D_OUT].set(b2)

  degp = _deg_call(dst3)
  y1 = _mm1_call(x, W1, degp)
  p1 = _edge_call(y1, src3, dst3)
  y2 = _mm2_call(p1, degp, w2p, b1r)
  p2 = _edge_call(y2, src3, dst3)
  return _post_call(p2, degp, b2p)


# speedup vs baseline: 1.1931x; 1.1931x over previous
import functools

import jax
import jax.numpy as jnp
from jax import lax
from jax.experimental import pallas as pl
from jax.experimental.pallas import tpu as pltpu
from jax.experimental.pallas import tpu_sc as plsc

N_NODES = 10000
N_EDGES = 160000
D_IN = 767
D_H = 16
D_OUT = 10

NODES_P = 10240
E_TOT = N_EDGES + N_NODES
NW = 32
CH = 128
EPW = 5376
E_PAD = EPW * NW
NCH = EPW // CH
RPS = NODES_P // 16

ROW_BLK = 2048
N_BLKS = NODES_P // ROW_BLK

_sc_mesh = functools.partial(
    plsc.VectorSubcoreMesh, core_axis_name="c", subcore_axis_name="s")
_sc_params = pltpu.CompilerParams(use_tc_tiling_on_sc=False)


def _deg_body(dst_hbm, outd_hbm, acc, zbuf, ones, didx, isem, ssem, csem):
  cid = lax.axis_index("c")
  sid = lax.axis_index("s")
  wid = sid * 2 + cid
  pltpu.async_copy(dst_hbm.at[wid], didx, isem)
  zero16 = jnp.zeros((16,), jnp.float32)
  one16 = jnp.ones((16,), jnp.float32)
  for i in range(RPS // 16):
    zbuf[pl.ds(i * 16, 16)] = zero16
  for i in range(CH // 16):
    ones[pl.ds(i * 16, 16)] = one16
  pltpu.sync_copy(zbuf, acc.at[pl.ds(sid * RPS, RPS)])
  pltpu.make_async_copy(dst_hbm.at[wid], didx, isem).wait()
  plsc.subcore_barrier()

  def fire(k, carry):
    pltpu.async_copy(ones, acc.at[didx.at[k]], ssem, add=True)
    return carry

  lax.fori_loop(0, NCH, fire, 0)

  def drain(k, carry):
    pltpu.make_async_copy(ones, acc.at[didx.at[k]], ssem).wait()
    return carry

  lax.fori_loop(0, NCH, drain, 0)
  plsc.subcore_barrier()
  pltpu.async_copy(acc.at[pl.ds(sid * RPS, RPS)],
                   outd_hbm.at[pl.ds(cid * NODES_P + sid * RPS, RPS)],
                   csem).wait()


def _deg_call(dst3):
  return pl.kernel(
      _deg_body,
      out_type=jax.ShapeDtypeStruct((2 * NODES_P,), jnp.float32),
      mesh=_sc_mesh(),
      compiler_params=_sc_params,
      scratch_types=[
          pltpu.VMEM_SHARED((NODES_P,), jnp.float32),
          pltpu.VMEM((RPS,), jnp.float32),
          pltpu.VMEM((CH,), jnp.float32),
          pltpu.VMEM((NCH, CH), jnp.int32),
          pltpu.SemaphoreType.DMA,
          pltpu.SemaphoreType.DMA,
          pltpu.SemaphoreType.DMA,
      ],
  )(dst3)


_NB = 7
_LAG = 3


def _edge_body(y_hbm, src_hbm, dst_hbm, out_hbm, acc, zbuf, sidx, didx,
               *rest):
  rows = list(rest[:_NB])
  gsem = list(rest[_NB:2 * _NB])
  ssem = list(rest[2 * _NB:3 * _NB])
  isem, zsem, csem = rest[3 * _NB:]
  cid = lax.axis_index("c")
  sid = lax.axis_index("s")
  wid = sid * 2 + cid
  pltpu.async_copy(src_hbm.at[wid], sidx, isem)
  pltpu.async_copy(dst_hbm.at[wid], didx, isem)
  zero16 = jnp.zeros((16,), jnp.float32)
  for i in range(64):
    zbuf[i, :] = zero16
  for k in range(RPS // 64):
    pltpu.async_copy(zbuf, acc.at[pl.ds(sid * RPS + k * 64, 64)], zsem)
  for k in range(RPS // 64):
    pltpu.make_async_copy(zbuf, acc.at[pl.ds(sid * RPS + k * 64, 64)],
                          zsem).wait()
  pltpu.make_async_copy(src_hbm.at[wid], sidx, isem).wait()
  pltpu.make_async_copy(dst_hbm.at[wid], didx, isem).wait()
  plsc.subcore_barrier()

  def gather(kk, b):
    pltpu.async_copy(y_hbm.at[sidx.at[kk]], rows[b], gsem[b])

  def wait_gather(kk, b):
    pltpu.make_async_copy(y_hbm.at[sidx.at[kk]], rows[b], gsem[b]).wait()

  def scatter(kk, b):
    pltpu.async_copy(rows[b], acc.at[didx.at[kk]], ssem[b], add=True)

  def wait_scatter(kk, b):
    pltpu.make_async_copy(rows[b], acc.at[didx.at[kk]], ssem[b]).wait()

  for kk in range(_NB):
    gather(kk, kk)
    if kk >= _LAG:
      wait_gather(kk - _LAG, kk - _LAG)
      scatter(kk - _LAG, kk - _LAG)

  def steady(i, carry):
    kbase = i * _NB
    for b in range(_NB):
      kk = kbase + b
      wait_scatter(kk - _NB, b)
      gather(kk, b)
      bj = (b - _LAG) % _NB
      wait_gather(kk - _LAG, bj)
      scatter(kk - _LAG, bj)
    return carry

  lax.fori_loop(1, NCH // _NB, steady, 0)

  for j in range(NCH - _LAG, NCH):
    wait_gather(j, j % _NB)
    scatter(j, j % _NB)
  for kk in range(NCH - _NB, NCH):
    wait_scatter(kk, kk % _NB)

  plsc.subcore_barrier()
  pltpu.async_copy(acc.at[pl.ds(sid * RPS, RPS)],
                   out_hbm.at[cid, pl.ds(sid * RPS, RPS)], csem).wait()


def _edge_call(y_pad, src3, dst3):
  return pl.kernel(
      _edge_body,
      out_type=jax.ShapeDtypeStruct((2, NODES_P, D_H), jnp.float32),
      mesh=_sc_mesh(),
      compiler_params=_sc_params,
      scratch_types=(
          [pltpu.VMEM_SHARED((NODES_P, D_H), jnp.float32),
           pltpu.VMEM((64, D_H), jnp.float32),
           pltpu.VMEM((NCH, CH), jnp.int32),
           pltpu.VMEM((NCH, CH), jnp.int32)]
          + [pltpu.VMEM((CH, D_H), jnp.float32)] * _NB
          + [pltpu.SemaphoreType.DMA] * (2 * _NB + 3)
      ),
  )(y_pad, src3, dst3)


def _dinv_flat(degp_ref, lo, size):
  deg = (degp_ref[pl.ds(lo, size)] + degp_ref[pl.ds(NODES_P + lo, size)])
  return jnp.where(deg > 0, lax.rsqrt(deg), 0.0)


def _mm1_body(x_ref, w_ref, degp_ref, y_ref):
  i = pl.program_id(0)
  xw = jnp.dot(x_ref[...], w_ref[...], preferred_element_type=jnp.float32)
  dinv = _dinv_flat(degp_ref, i * ROW_BLK, ROW_BLK)
  y_ref[...] = xw * dinv[:, None]


def _mm1_call(x, w1, degp):
  return pl.pallas_call(
      _mm1_body,
      grid=(N_BLKS,),
      in_specs=[
          pl.BlockSpec((ROW_BLK, D_IN), lambda i: (i, 0)),
          pl.BlockSpec((D_IN, D_H), lambda i: (0, 0)),
          pl.BlockSpec((2 * NODES_P,), lambda i: (0,)),
      ],
      out_specs=pl.BlockSpec((ROW_BLK, D_H), lambda i: (i, 0)),
      out_shape=jax.ShapeDtypeStruct((NODES_P, D_H), jnp.float32),
  )(x, w1, degp)


NP8 = NODES_P // 8


def _dinv_packed(degb_ref):
  degb = degb_ref[0] + degb_ref[1]
  return jnp.where(degb > 0, lax.rsqrt(degb), 0.0)


def _mm2_body(p_ref, degb_ref, w2bd_ref, b1_ref, y_ref):
  dinv = _dinv_packed(degb_ref)
  acc = p_ref[0] + p_ref[1]
  h = jax.nn.relu(acc * dinv + b1_ref[0, :])
  y_ref[...] = jnp.dot(h, w2bd_ref[...],
                       preferred_element_type=jnp.float32) * dinv


def _mm2_call(p, degb, w2bd, b1b):
  return pl.pallas_call(
      _mm2_body,
      in_specs=[
          pl.BlockSpec((2, NP8, 128), lambda: (0, 0, 0)),
          pl.BlockSpec((2, NP8, 128), lambda: (0, 0, 0)),
          pl.BlockSpec((128, 128), lambda: (0, 0)),
          pl.BlockSpec((1, 128), lambda: (0, 0)),
      ],
      out_specs=pl.BlockSpec((NP8, 128), lambda: (0, 0)),
      out_shape=jax.ShapeDtypeStruct((NP8, 128), jnp.float32),
  )(p, degb, w2bd, b1b)


def _post_body(q_ref, degb_ref, b2_ref, out_ref):
  dinv = _dinv_packed(degb_ref)
  out_ref[...] = (q_ref[0] + q_ref[1]) * dinv + b2_ref[0, :]


def _post_call(q, degb, b2b):
  return pl.pallas_call(
      _post_body,
      in_specs=[
          pl.BlockSpec((2, NP8, 128), lambda: (0, 0, 0)),
          pl.BlockSpec((2, NP8, 128), lambda: (0, 0, 0)),
          pl.BlockSpec((1, 128), lambda: (0, 0)),
      ],
      out_specs=pl.BlockSpec((NP8, 128), lambda: (0, 0)),
      out_shape=jax.ShapeDtypeStruct((NP8, 128), jnp.float32),
  )(q, degb, b2b)


def kernel(x, edge_index, W1, b1, W2, b2):
  n = x.shape[0]
  loop = jnp.arange(n, dtype=jnp.int32)
  ed = jnp.concatenate(
      [edge_index,
       jnp.tile(loop[None], (2, 1)),
       jnp.full((2, E_PAD - E_TOT), n, dtype=jnp.int32)],
      axis=1).reshape(2, NW, NCH, CH)
  src3, dst3 = ed[0], ed[1]

  w2p = jnp.zeros((D_H, D_H), jnp.float32).at[:, :D_OUT].set(W2)
  w2bd = jnp.kron(jnp.eye(8, dtype=jnp.float32), w2p)
  b1b = jnp.tile(b1, 8).reshape(1, 128)
  b2p = jnp.zeros((D_H,), jnp.float32).at[:D_OUT].set(b2)
  b2b = jnp.tile(b2p, 8).reshape(1, 128)

  degp = _deg_call(dst3)
  degbp = jnp.broadcast_to(degp.reshape(2, NODES_P, 1),
                           (2, NODES_P, D_H)).reshape(2, NP8, 128)
  y1 = _mm1_call(x, W1, degp)
  p1 = _edge_call(y1, src3, dst3)
  y2 = _mm2_call(p1.reshape(2, NP8, 128), degbp, w2bd, b1b)
  p2 = _edge_call(y2.reshape(NODES_P, D_H), src3, dst3)
  res = _post_call(p2.reshape(2, NP8, 128), degbp, b2b)
  return res.reshape(NODES_P, D_H)[:n, :D_OUT]


# speedup vs baseline: 1.4371x; 1.2045x over previous
import functools

import jax
import jax.numpy as jnp
from jax import lax
from jax.experimental import pallas as pl
from jax.experimental.pallas import tpu as pltpu
from jax.experimental.pallas import tpu_sc as plsc

N_NODES = 10000
N_EDGES = 160000
D_IN = 767
D_H = 16
D_OUT = 10

NODES_P = 10240
E_TOT = N_EDGES + N_NODES
NW = 32
CH = 128
EPW = 5376
E_PAD = EPW * NW
NCH = EPW // CH
RPS = NODES_P // 16

ROW_BLK = 2048
N_BLKS = NODES_P // ROW_BLK

_sc_mesh = functools.partial(
    plsc.VectorSubcoreMesh, core_axis_name="c", subcore_axis_name="s")
_sc_params = pltpu.CompilerParams(use_tc_tiling_on_sc=False)


def _deg_body(dst_hbm, outd_hbm, acc, zbuf, ones, didx, isem, ssem, csem):
  cid = lax.axis_index("c")
  sid = lax.axis_index("s")
  wid = sid * 2 + cid
  pltpu.async_copy(dst_hbm.at[wid], didx, isem)
  zero16 = jnp.zeros((16,), jnp.float32)
  one16 = jnp.ones((16,), jnp.float32)
  for i in range(RPS // 16):
    zbuf[pl.ds(i * 16, 16)] = zero16
  for i in range(CH // 16):
    ones[pl.ds(i * 16, 16)] = one16
  pltpu.sync_copy(zbuf, acc.at[pl.ds(sid * RPS, RPS)])
  pltpu.make_async_copy(dst_hbm.at[wid], didx, isem).wait()
  plsc.subcore_barrier()

  def fire(k, carry):
    pltpu.async_copy(ones, acc.at[didx.at[k]], ssem, add=True)
    return carry

  lax.fori_loop(0, NCH, fire, 0)

  def drain(k, carry):
    pltpu.make_async_copy(ones, acc.at[didx.at[k]], ssem).wait()
    return carry

  lax.fori_loop(0, NCH, drain, 0)
  plsc.subcore_barrier()
  pltpu.async_copy(acc.at[pl.ds(sid * RPS, RPS)],
                   outd_hbm.at[pl.ds(cid * NODES_P + sid * RPS, RPS)],
                   csem).wait()


def _deg_call(dst3):
  return pl.kernel(
      _deg_body,
      out_type=jax.ShapeDtypeStruct((2 * NODES_P,), jnp.float32),
      mesh=_sc_mesh(),
      compiler_params=_sc_params,
      scratch_types=[
          pltpu.VMEM_SHARED((NODES_P,), jnp.float32),
          pltpu.VMEM((RPS,), jnp.float32),
          pltpu.VMEM((CH,), jnp.float32),
          pltpu.VMEM((NCH, CH), jnp.int32),
          pltpu.SemaphoreType.DMA,
          pltpu.SemaphoreType.DMA,
          pltpu.SemaphoreType.DMA,
      ],
  )(dst3)


_NB = 7
_LAG = 3


def _edge_body(y_hbm, src_hbm, dst_hbm, out_hbm, acc, ysh, zbuf, sidx, didx,
               *rest):
  rows = list(rest[:_NB])
  gsem = list(rest[_NB:2 * _NB])
  ssem = list(rest[2 * _NB:3 * _NB])
  isem, zsem, csem = rest[3 * _NB:]
  cid = lax.axis_index("c")
  sid = lax.axis_index("s")
  wid = sid * 2 + cid
  pltpu.async_copy(src_hbm.at[wid], sidx, isem)
  pltpu.async_copy(dst_hbm.at[wid], didx, isem)
  pltpu.async_copy(y_hbm.at[pl.ds(sid * RPS, RPS)],
                   ysh.at[pl.ds(sid * RPS, RPS)], isem)
  zero16 = jnp.zeros((16,), jnp.float32)
  for i in range(64):
    zbuf[i, :] = zero16
  for k in range(RPS // 64):
    pltpu.async_copy(zbuf, acc.at[pl.ds(sid * RPS + k * 64, 64)], zsem)
  for k in range(RPS // 64):
    pltpu.make_async_copy(zbuf, acc.at[pl.ds(sid * RPS + k * 64, 64)],
                          zsem).wait()
  pltpu.make_async_copy(src_hbm.at[wid], sidx, isem).wait()
  pltpu.make_async_copy(dst_hbm.at[wid], didx, isem).wait()
  pltpu.make_async_copy(y_hbm.at[pl.ds(sid * RPS, RPS)],
                        ysh.at[pl.ds(sid * RPS, RPS)], isem).wait()
  plsc.subcore_barrier()

  def gather(kk, b):
    pltpu.async_copy(ysh.at[sidx.at[kk]], rows[b], gsem[b])

  def wait_gather(kk, b):
    pltpu.make_async_copy(ysh.at[sidx.at[kk]], rows[b], gsem[b]).wait()

  def scatter(kk, b):
    pltpu.async_copy(rows[b], acc.at[didx.at[kk]], ssem[b], add=True)

  def wait_scatter(kk, b):
    pltpu.make_async_copy(rows[b], acc.at[didx.at[kk]], ssem[b]).wait()

  for kk in range(_NB):
    gather(kk, kk)
    if kk >= _LAG:
      wait_gather(kk - _LAG, kk - _LAG)
      scatter(kk - _LAG, kk - _LAG)

  def steady(i, carry):
    kbase = i * _NB
    for b in range(_NB):
      kk = kbase + b
      wait_scatter(kk - _NB, b)
      gather(kk, b)
      bj = (b - _LAG) % _NB
      wait_gather(kk - _LAG, bj)
      scatter(kk - _LAG, bj)
    return carry

  lax.fori_loop(1, NCH // _NB, steady, 0)

  for j in range(NCH - _LAG, NCH):
    wait_gather(j, j % _NB)
    scatter(j, j % _NB)
  for kk in range(NCH - _NB, NCH):
    wait_scatter(kk, kk % _NB)

  plsc.subcore_barrier()
  pltpu.async_copy(acc.at[pl.ds(sid * RPS, RPS)],
                   out_hbm.at[cid, pl.ds(sid * RPS, RPS)], csem).wait()


def _edge_call(y_pad, src3, dst3):
  return pl.kernel(
      _edge_body,
      out_type=jax.ShapeDtypeStruct((2, NODES_P, D_H), jnp.float32),
      mesh=_sc_mesh(),
      compiler_params=_sc_params,
      scratch_types=(
          [pltpu.VMEM_SHARED((NODES_P, D_H), jnp.float32),
           pltpu.VMEM_SHARED((NODES_P, D_H), jnp.float32),
           pltpu.VMEM((64, D_H), jnp.float32),
           pltpu.VMEM((NCH, CH), jnp.int32),
           pltpu.VMEM((NCH, CH), jnp.int32)]
          + [pltpu.VMEM((CH, D_H), jnp.float32)] * _NB
          + [pltpu.SemaphoreType.DMA] * (2 * _NB + 3)
      ),
  )(y_pad, src3, dst3)


def _dinv_flat(degp_ref, lo, size):
  deg = (degp_ref[pl.ds(lo, size)] + degp_ref[pl.ds(NODES_P + lo, size)])
  return jnp.where(deg > 0, lax.rsqrt(deg), 0.0)


def _mm1_body(x_ref, w_ref, degp_ref, y_ref):
  i = pl.program_id(0)
  xw = jnp.dot(x_ref[...], w_ref[...], preferred_element_type=jnp.float32)
  dinv = _dinv_flat(degp_ref, i * ROW_BLK, ROW_BLK)
  y_ref[...] = xw * dinv[:, None]


def _mm1_call(x, w1, degp):
  return pl.pallas_call(
      _mm1_body,
      grid=(N_BLKS,),
      in_specs=[
          pl.BlockSpec((ROW_BLK, D_IN), lambda i: (i, 0)),
          pl.BlockSpec((D_IN, D_H), lambda i: (0, 0)),
          pl.BlockSpec((2 * NODES_P,), lambda i: (0,)),
      ],
      out_specs=pl.BlockSpec((ROW_BLK, D_H), lambda i: (i, 0)),
      out_shape=jax.ShapeDtypeStruct((NODES_P, D_H), jnp.float32),
  )(x, w1, degp)


NP8 = NODES_P // 8


def _dinv_packed(degb_ref):
  degb = degb_ref[0] + degb_ref[1]
  return jnp.where(degb > 0, lax.rsqrt(degb), 0.0)


def _mm2_body(p_ref, degb_ref, w2bd_ref, b1_ref, y_ref):
  dinv = _dinv_packed(degb_ref)
  acc = p_ref[0] + p_ref[1]
  h = jax.nn.relu(acc * dinv + b1_ref[0, :])
  y_ref[...] = jnp.dot(h, w2bd_ref[...],
                       preferred_element_type=jnp.float32) * dinv


def _mm2_call(p, degb, w2bd, b1b):
  return pl.pallas_call(
      _mm2_body,
      in_specs=[
          pl.BlockSpec((2, NP8, 128), lambda: (0, 0, 0)),
          pl.BlockSpec((2, NP8, 128), lambda: (0, 0, 0)),
          pl.BlockSpec((128, 128), lambda: (0, 0)),
          pl.BlockSpec((1, 128), lambda: (0, 0)),
      ],
      out_specs=pl.BlockSpec((NP8, 128), lambda: (0, 0)),
      out_shape=jax.ShapeDtypeStruct((NP8, 128), jnp.float32),
  )(p, degb, w2bd, b1b)


def _post_body(q_ref, degb_ref, b2_ref, out_ref):
  dinv = _dinv_packed(degb_ref)
  out_ref[...] = (q_ref[0] + q_ref[1]) * dinv + b2_ref[0, :]


def _post_call(q, degb, b2b):
  return pl.pallas_call(
      _post_body,
      in_specs=[
          pl.BlockSpec((2, NP8, 128), lambda: (0, 0, 0)),
          pl.BlockSpec((2, NP8, 128), lambda: (0, 0, 0)),
          pl.BlockSpec((1, 128), lambda: (0, 0)),
      ],
      out_specs=pl.BlockSpec((NP8, 128), lambda: (0, 0)),
      out_shape=jax.ShapeDtypeStruct((NP8, 128), jnp.float32),
  )(q, degb, b2b)


def kernel(x, edge_index, W1, b1, W2, b2):
  n = x.shape[0]
  loop = jnp.arange(n, dtype=jnp.int32)
  ed = jnp.concatenate(
      [edge_index,
       jnp.tile(loop[None], (2, 1)),
       jnp.full((2, E_PAD - E_TOT), n, dtype=jnp.int32)],
      axis=1).reshape(2, NW, NCH, CH)
  src3, dst3 = ed[0], ed[1]

  w2p = jnp.zeros((D_H, D_H), jnp.float32).at[:, :D_OUT].set(W2)
  w2bd = jnp.kron(jnp.eye(8, dtype=jnp.float32), w2p)
  b1b = jnp.tile(b1, 8).reshape(1, 128)
  b2p = jnp.zeros((D_H,), jnp.float32).at[:D_OUT].set(b2)
  b2b = jnp.tile(b2p, 8).reshape(1, 128)

  degp = _deg_call(dst3)
  degbp = jnp.broadcast_to(degp.reshape(2, NODES_P, 1),
                           (2, NODES_P, D_H)).reshape(2, NP8, 128)
  y1 = _mm1_call(x, W1, degp)
  p1 = _edge_call(y1, src3, dst3)
  y2 = _mm2_call(p1.reshape(2, NP8, 128), degbp, w2bd, b1b)
  p2 = _edge_call(y2.reshape(NODES_P, D_H), src3, dst3)
  res = _post_call(p2.reshape(2, NP8, 128), degbp, b2b)
  return res.reshape(NODES_P, D_H)[:n, :D_OUT]


# speedup vs baseline: 1.4952x; 1.0404x over previous
import functools

import jax
import jax.numpy as jnp
from jax import lax
from jax.experimental import pallas as pl
from jax.experimental.pallas import tpu as pltpu
from jax.experimental.pallas import tpu_sc as plsc

N_NODES = 10000
N_EDGES = 160000
D_IN = 767
D_H = 16
D_OUT = 10

NODES_P = 10240
E_TOT = N_EDGES + N_NODES
NW = 32
CH = 128
EPW = 5376
E_PAD = EPW * NW
NCH = EPW // CH
RPS = NODES_P // 16

ROW_BLK = 2048
N_BLKS = NODES_P // ROW_BLK

_sc_mesh = functools.partial(
    plsc.VectorSubcoreMesh, core_axis_name="c", subcore_axis_name="s")
_sc_params = pltpu.CompilerParams(use_tc_tiling_on_sc=False)


def _deg_body(dst_hbm, outd_hbm, acc, zbuf, ones, didx, isem, ssem, csem):
  cid = lax.axis_index("c")
  sid = lax.axis_index("s")
  wid = sid * 2 + cid
  pltpu.async_copy(dst_hbm.at[wid], didx, isem)
  zero16 = jnp.zeros((16,), jnp.float32)
  one16 = jnp.ones((16,), jnp.float32)
  for i in range(RPS // 16):
    zbuf[pl.ds(i * 16, 16)] = zero16
  for i in range(CH // 16):
    ones[pl.ds(i * 16, 16)] = one16
  pltpu.sync_copy(zbuf, acc.at[pl.ds(sid * RPS, RPS)])
  pltpu.make_async_copy(dst_hbm.at[wid], didx, isem).wait()
  plsc.subcore_barrier()

  def fire(k, carry):
    pltpu.async_copy(ones, acc.at[didx.at[k]], ssem, add=True)
    return carry

  lax.fori_loop(0, NCH, fire, 0)

  def drain(k, carry):
    pltpu.make_async_copy(ones, acc.at[didx.at[k]], ssem).wait()
    return carry

  lax.fori_loop(0, NCH, drain, 0)
  plsc.subcore_barrier()
  pltpu.async_copy(acc.at[pl.ds(sid * RPS, RPS)],
                   outd_hbm.at[pl.ds(cid * NODES_P + sid * RPS, RPS)],
                   csem).wait()


def _deg_call(dst3):
  return pl.kernel(
      _deg_body,
      out_type=jax.ShapeDtypeStruct((2 * NODES_P,), jnp.float32),
      mesh=_sc_mesh(),
      compiler_params=_sc_params,
      scratch_types=[
          pltpu.VMEM_SHARED((NODES_P,), jnp.float32),
          pltpu.VMEM((RPS,), jnp.float32),
          pltpu.VMEM((CH,), jnp.float32),
          pltpu.VMEM((NCH, CH), jnp.int32),
          pltpu.SemaphoreType.DMA,
          pltpu.SemaphoreType.DMA,
          pltpu.SemaphoreType.DMA,
      ],
  )(dst3)


_NB = 7
_LAG = 3


def _edge_body(y_hbm, src_hbm, dst_hbm, out_hbm, acc, ysh, zbuf, sidx, didx,
               *rest):
  rows = list(rest[:_NB])
  gsem = list(rest[_NB:2 * _NB])
  ssem = list(rest[2 * _NB:3 * _NB])
  isem, zsem, csem = rest[3 * _NB:]
  cid = lax.axis_index("c")
  sid = lax.axis_index("s")
  wid = sid * 2 + cid
  pltpu.async_copy(src_hbm.at[wid], sidx, isem)
  pltpu.async_copy(dst_hbm.at[wid], didx, isem)
  pltpu.async_copy(y_hbm.at[pl.ds(sid * RPS, RPS)],
                   ysh.at[pl.ds(sid * RPS, RPS)], isem)
  zero16 = jnp.zeros((16,), jnp.float32)
  for i in range(64):
    zbuf[i, :] = zero16
  for k in range(RPS // 64):
    pltpu.async_copy(zbuf, acc.at[pl.ds(sid * RPS + k * 64, 64)], zsem)
  for k in range(RPS // 64):
    pltpu.make_async_copy(zbuf, acc.at[pl.ds(sid * RPS + k * 64, 64)],
                          zsem).wait()
  pltpu.make_async_copy(src_hbm.at[wid], sidx, isem).wait()
  pltpu.make_async_copy(dst_hbm.at[wid], didx, isem).wait()
  pltpu.make_async_copy(y_hbm.at[pl.ds(sid * RPS, RPS)],
                        ysh.at[pl.ds(sid * RPS, RPS)], isem).wait()
  plsc.subcore_barrier()

  def gather(kk, b):
    pltpu.async_copy(ysh.at[sidx.at[kk]], rows[b], gsem[b])

  def wait_gather(kk, b):
    pltpu.make_async_copy(ysh.at[sidx.at[kk]], rows[b], gsem[b]).wait()

  def scatter(kk, b):
    pltpu.async_copy(rows[b], acc.at[didx.at[kk]], ssem[b], add=True)

  def wait_scatter(kk, b):
    pltpu.make_async_copy(rows[b], acc.at[didx.at[kk]], ssem[b]).wait()

  for kk in range(_NB):
    gather(kk, kk)
    if kk >= _LAG:
      wait_gather(kk - _LAG, kk - _LAG)
      scatter(kk - _LAG, kk - _LAG)

  def steady(i, carry):
    kbase = i * _NB
    for b in range(_NB):
      kk = kbase + b
      wait_scatter(kk - _NB, b)
      gather(kk, b)
      bj = (b - _LAG) % _NB
      wait_gather(kk - _LAG, bj)
      scatter(kk - _LAG, bj)
    return carry

  lax.fori_loop(1, NCH // _NB, steady, 0)

  for j in range(NCH - _LAG, NCH):
    wait_gather(j, j % _NB)
    scatter(j, j % _NB)
  for kk in range(NCH - _NB, NCH):
    wait_scatter(kk, kk % _NB)

  plsc.subcore_barrier()
  pltpu.async_copy(acc.at[pl.ds(sid * RPS, RPS)],
                   out_hbm.at[cid, pl.ds(sid * RPS, RPS)], csem).wait()


def _edge_call(y_pad, src3, dst3):
  return pl.kernel(
      _edge_body,
      out_type=jax.ShapeDtypeStruct((2, NODES_P, D_H), jnp.float32),
      mesh=_sc_mesh(),
      compiler_params=_sc_params,
      scratch_types=(
          [pltpu.VMEM_SHARED((NODES_P, D_H), jnp.float32),
           pltpu.VMEM_SHARED((NODES_P, D_H), jnp.float32),
           pltpu.VMEM((64, D_H), jnp.float32),
           pltpu.VMEM((NCH, CH), jnp.int32),
           pltpu.VMEM((NCH, CH), jnp.int32)]
          + [pltpu.VMEM((CH, D_H), jnp.float32)] * _NB
          + [pltpu.SemaphoreType.DMA] * (2 * _NB + 3)
      ),
  )(y_pad, src3, dst3)


def _dinv_flat(degp_ref, lo, size):
  deg = (degp_ref[pl.ds(lo, size)] + degp_ref[pl.ds(NODES_P + lo, size)])
  return jnp.where(deg > 0, lax.rsqrt(deg), 0.0)


NP8 = NODES_P // 8


def _mm1_body(x_ref, w_ref, degp_ref, deg8_ref, m8_ref, y_ref, dinv_ref):
  i = pl.program_id(0)
  xw = jnp.dot(x_ref[...], w_ref[...], preferred_element_type=jnp.float32)
  dinv = _dinv_flat(degp_ref, i * ROW_BLK, ROW_BLK)
  y_ref[...] = xw * dinv[:, None]

  @pl.when(i == 0)
  def _():
    d8 = deg8_ref[pl.ds(0, NP8), :] + deg8_ref[pl.ds(NP8, NP8), :]
    deg128 = jnp.dot(d8, m8_ref[...], preferred_element_type=jnp.float32)
    dinv_ref[...] = jnp.where(deg128 > 0, lax.rsqrt(deg128), 0.0)


def _mm1_call(x, w1, degp, deg8, m8):
  return pl.pallas_call(
      _mm1_body,
      grid=(N_BLKS,),
      in_specs=[
          pl.BlockSpec((ROW_BLK, D_IN), lambda i: (i, 0)),
          pl.BlockSpec((D_IN, D_H), lambda i: (0, 0)),
          pl.BlockSpec((2 * NODES_P,), lambda i: (0,)),
          pl.BlockSpec((2 * NP8, 8), lambda i: (0, 0)),
          pl.BlockSpec((8, 128), lambda i: (0, 0)),
      ],
      out_specs=[
          pl.BlockSpec((ROW_BLK, D_H), lambda i: (i, 0)),
          pl.BlockSpec((NP8, 128), lambda i: (0, 0)),
      ],
      out_shape=[
          jax.ShapeDtypeStruct((NODES_P, D_H), jnp.float32),
          jax.ShapeDtypeStruct((NP8, 128), jnp.float32),
      ],
  )(x, w1, degp, deg8, m8)


def _mm2_body(p_ref, dinv_ref, w2bd_ref, b1_ref, y_ref):
  dinv = dinv_ref[...]
  acc = p_ref[0] + p_ref[1]
  h = jax.nn.relu(acc * dinv + b1_ref[0, :])
  y_ref[...] = jnp.dot(h, w2bd_ref[...],
                       preferred_element_type=jnp.float32) * dinv


def _mm2_call(p, dinv128, w2bd, b1b):
  return pl.pallas_call(
      _mm2_body,
      in_specs=[
          pl.BlockSpec((2, NP8, 128), lambda: (0, 0, 0)),
          pl.BlockSpec((NP8, 128), lambda: (0, 0)),
          pl.BlockSpec((128, 128), lambda: (0, 0)),
          pl.BlockSpec((1, 128), lambda: (0, 0)),
      ],
      out_specs=pl.BlockSpec((NP8, 128), lambda: (0, 0)),
      out_shape=jax.ShapeDtypeStruct((NP8, 128), jnp.float32),
  )(p, dinv128, w2bd, b1b)


def _post_body(q_ref, dinv_ref, b2_ref, out_ref):
  out_ref[...] = (q_ref[0] + q_ref[1]) * dinv_ref[...] + b2_ref[0, :]


def _post_call(q, dinv128, b2b):
  return pl.pallas_call(
      _post_body,
      in_specs=[
          pl.BlockSpec((2, NP8, 128), lambda: (0, 0, 0)),
          pl.BlockSpec((NP8, 128), lambda: (0, 0)),
          pl.BlockSpec((1, 128), lambda: (0, 0)),
      ],
      out_specs=pl.BlockSpec((NP8, 128), lambda: (0, 0)),
      out_shape=jax.ShapeDtypeStruct((NP8, 128), jnp.float32),
  )(q, dinv128, b2b)


def kernel(x, edge_index, W1, b1, W2, b2):
  n = x.shape[0]
  loop = jnp.arange(n, dtype=jnp.int32)
  ed = jnp.concatenate(
      [edge_index,
       jnp.tile(loop[None], (2, 1)),
       jnp.full((2, E_PAD - E_TOT), n, dtype=jnp.int32)],
      axis=1).reshape(2, NW, NCH, CH)
  src3, dst3 = ed[0], ed[1]

  w2p = jnp.zeros((D_H, D_H), jnp.float32).at[:, :D_OUT].set(W2)
  w2bd = jnp.kron(jnp.eye(8, dtype=jnp.float32), w2p)
  b1b = jnp.tile(b1, 8).reshape(1, 128)
  b2p = jnp.zeros((D_H,), jnp.float32).at[:D_OUT].set(b2)
  b2b = jnp.tile(b2p, 8).reshape(1, 128)

  m8 = jnp.repeat(jnp.eye(8, dtype=jnp.float32), D_H, axis=1)

  degp = _deg_call(dst3)
  deg8 = degp.reshape(2 * NP8, 8)
  y1, dinv128 = _mm1_call(x, W1, degp, deg8, m8)
  p1 = _edge_call(y1, src3, dst3)
  y2 = _mm2_call(p1.reshape(2, NP8, 128), dinv128, w2bd, b1b)
  p2 = _edge_call(y2.reshape(NODES_P, D_H), src3, dst3)
  res = _post_call(p2.reshape(2, NP8, 128), dinv128, b2b)
  return res.reshape(NODES_P, D_H)[:n, :D_OUT]
